# baseline (speedup 1.0000x reference)
import jax
import jax.numpy as jnp
from jax.experimental import pallas as pl
from jax.experimental.pallas import tpu as pltpu

BLK_T = 512


def _mlp_block(x_ref, wg_ref, wu_ref, wd_ref, o_ref):
    xb = x_ref[...]
    gate = jnp.dot(xb, wg_ref[...], preferred_element_type=jnp.float32)
    up = jnp.dot(xb, wu_ref[...], preferred_element_type=jnp.float32)
    hidden = jax.nn.silu(gate) * up
    o_ref[...] = jnp.dot(hidden, wd_ref[...], preferred_element_type=jnp.float32)


def kernel(x, W_gate, W_up, W_down):
    n_tokens, d_model = x.shape
    d_ff = W_gate.shape[1]
    grid = (n_tokens // BLK_T,)
    return pl.pallas_call(
        _mlp_block,
        grid=grid,
        in_specs=[
            pl.BlockSpec((BLK_T, d_model), lambda i: (i, 0)),
            pl.BlockSpec((d_model, d_ff), lambda i: (0, 0)),
            pl.BlockSpec((d_model, d_ff), lambda i: (0, 0)),
            pl.BlockSpec((d_ff, d_model), lambda i: (0, 0)),
        ],
        out_specs=pl.BlockSpec((BLK_T, d_model), lambda i: (i, 0)),
        out_shape=jax.ShapeDtypeStruct((n_tokens, d_model), jnp.float32),
        compiler_params=pltpu.CompilerParams(
            dimension_semantics=("arbitrary",),
        ),
    )(x, W_gate, W_up, W_down)
